# baseline (speedup 1.0000x reference)
import functools

import jax
import jax.numpy as jnp
from jax import lax
from jax.experimental import pallas as pl
from jax.experimental.pallas import tpu as pltpu
from jax.experimental.pallas import tpu_sc as plsc

_INFO = plsc.get_sparse_core_info()
_NC, _NS, _L = _INFO.num_cores, _INFO.num_subcores, _INFO.num_lanes
_NW = _NC * _NS


def _make_lookup(B, D):
    BPW = B // _NW
    IDX_SLICE = 128
    n_slices = BPW // IDX_SLICE
    mesh = plsc.VectorSubcoreMesh(core_axis_name="c", subcore_axis_name="s")

    @functools.partial(
        pl.kernel,
        mesh=mesh,
        compiler_params=pltpu.CompilerParams(
            use_tc_tiling_on_sc=False, needs_layout_passes=False),
        out_type=jax.ShapeDtypeStruct((B, D), jnp.float32),
        scratch_types=[
            pltpu.VMEM((BPW,), jnp.int32),
            pltpu.VMEM((BPW, D), jnp.float32),
            pltpu.SemaphoreType.DMA,
        ],
    )
    def lookup(table_hbm, idx_hbm, out_hbm, idx_v, rows_v, sem):
        wid = lax.axis_index("s") * _NC + lax.axis_index("c")
        base = wid * BPW
        pltpu.sync_copy(idx_hbm.at[pl.ds(base, BPW)], idx_v)
        copies = [
            pltpu.async_copy(
                table_hbm.at[idx_v.at[pl.ds(j * IDX_SLICE, IDX_SLICE)]],
                rows_v.at[pl.ds(j * IDX_SLICE, IDX_SLICE)],
                sem,
            )
            for j in range(n_slices)
        ]
        for c in copies:
            c.wait()

        zeros = jnp.zeros((_L,), jnp.float32)

        def fix_chunk(c, carry):
            ids = idx_v[pl.ds(c * _L, _L)]
            pad = ids == 0
            npad = jnp.sum(pad.astype(jnp.int32))

            @pl.when(npad > 0)
            def _():
                rows = c * _L + lax.iota(jnp.int32, _L)
                for col in range(D):
                    plsc.store_scatter(
                        rows_v,
                        [rows, jnp.full((_L,), col, jnp.int32)],
                        zeros,
                        mask=pad,
                    )

            return carry

        lax.fori_loop(0, BPW // _L, fix_chunk, 0)
        pltpu.sync_copy(rows_v, out_hbm.at[pl.ds(base, BPW)])

    return lookup


def kernel(inputs, embedding_matrix):
    nb, ctx = inputs.shape
    _, D = embedding_matrix.shape
    idx = inputs.reshape(-1).astype(jnp.int32)
    out = _make_lookup(nb * ctx, D)(embedding_matrix, idx)
    return out.reshape(nb, ctx, D)


# speedup vs baseline: 1.2465x; 1.2465x over previous
import functools

import jax
import jax.numpy as jnp
from jax import lax
from jax.experimental import pallas as pl
from jax.experimental.pallas import tpu as pltpu
from jax.experimental.pallas import tpu_sc as plsc

_INFO = plsc.get_sparse_core_info()
_NC, _NS, _L = _INFO.num_cores, _INFO.num_subcores, _INFO.num_lanes
_NW = _NC * _NS


def _make_lookup(B, D):
    BPW = B // _NW
    K = 16
    mesh = plsc.VectorSubcoreMesh(core_axis_name="c", subcore_axis_name="s")

    @functools.partial(
        pl.kernel,
        mesh=mesh,
        compiler_params=pltpu.CompilerParams(
            use_tc_tiling_on_sc=True, needs_layout_passes=False),
        out_type=jax.ShapeDtypeStruct((B, D), jnp.float32),
        scratch_types=[
            pltpu.VMEM((BPW,), jnp.int32),
            pltpu.VMEM((BPW, D), jnp.float32),
            pltpu.SemaphoreType.DMA,
        ],
    )
    def lookup(table_hbm, idx_hbm, out_hbm, idx_v, rows_v, sem):
        wid = lax.axis_index("s") * _NC + lax.axis_index("c")
        base = wid * BPW
        pltpu.sync_copy(idx_hbm.at[pl.ds(base, BPW)], idx_v)

        def gather_group(g, carry):
            chunk = idx_v[pl.ds(g * K, K)]
            for j in range(K):
                r = chunk[j]
                pltpu.async_copy(
                    table_hbm.at[pl.ds(r, 1)],
                    rows_v.at[pl.ds(g * K + j, 1)],
                    sem,
                )
            for j in range(K):
                pltpu.make_async_copy(
                    table_hbm.at[pl.ds(0, 1)],
                    rows_v.at[pl.ds(g * K + j, 1)],
                    sem,
                ).wait()
            return carry

        lax.fori_loop(0, BPW // K, gather_group, 0)

        zeros = jnp.zeros((_L,), jnp.float32)

        def fix_chunk(c, carry):
            ids = idx_v[pl.ds(c * _L, _L)]
            pad = ids == 0
            npad = jnp.sum(pad.astype(jnp.int32))

            @pl.when(npad > 0)
            def _():
                rows = c * _L + lax.iota(jnp.int32, _L)
                for col in range(D):
                    plsc.store_scatter(
                        rows_v,
                        [rows, jnp.full((_L,), col, jnp.int32)],
                        zeros,
                        mask=pad,
                    )

            return carry

        lax.fori_loop(0, BPW // _L, fix_chunk, 0)
        pltpu.sync_copy(rows_v, out_hbm.at[pl.ds(base, BPW)])

    return lookup


def kernel(inputs, embedding_matrix):
    nb, ctx = inputs.shape
    _, D = embedding_matrix.shape
    idx = inputs.reshape(-1).astype(jnp.int32)
    out = _make_lookup(nb * ctx, D)(embedding_matrix, idx)
    return out.reshape(nb, ctx, D)


# speedup vs baseline: 1.2667x; 1.0162x over previous
import functools

import jax
import jax.numpy as jnp
from jax import lax
from jax.experimental import pallas as pl
from jax.experimental.pallas import tpu as pltpu
from jax.experimental.pallas import tpu_sc as plsc

_INFO = plsc.get_sparse_core_info()
_NC, _NS, _L = _INFO.num_cores, _INFO.num_subcores, _INFO.num_lanes
_NW = _NC * _NS


def _make_lookup(NB, CTX, D):
    B = NB * CTX
    BPW = B // _NW
    WPB = _NW // NB
    K = 16
    mesh = plsc.VectorSubcoreMesh(core_axis_name="c", subcore_axis_name="s")

    @functools.partial(
        pl.kernel,
        mesh=mesh,
        compiler_params=pltpu.CompilerParams(
            use_tc_tiling_on_sc=True, needs_layout_passes=False),
        out_type=jax.ShapeDtypeStruct((NB, CTX, D), jnp.float32),
        scratch_types=[
            pltpu.VMEM((BPW,), jnp.int32),
            pltpu.VMEM((BPW, D), jnp.float32),
            pltpu.SemaphoreType.DMA,
        ],
    )
    def lookup(table_hbm, idx_hbm, out_hbm, idx_v, rows_v, sem):
        wid = lax.axis_index("s") * _NC + lax.axis_index("c")
        b = wid // WPB
        t0 = (wid % WPB) * BPW
        pltpu.sync_copy(idx_hbm.at[b, pl.ds(t0, BPW)], idx_v)

        def gather_group(g, carry):
            chunk = idx_v[pl.ds(g * K, K)]
            for j in range(K):
                r = chunk[j]
                pltpu.async_copy(
                    table_hbm.at[pl.ds(r, 1)],
                    rows_v.at[pl.ds(g * K + j, 1)],
                    sem,
                )
            for j in range(K):
                pltpu.make_async_copy(
                    table_hbm.at[pl.ds(0, 1)],
                    rows_v.at[pl.ds(g * K + j, 1)],
                    sem,
                ).wait()
            return carry

        lax.fori_loop(0, BPW // K, gather_group, 0)

        zeros = jnp.zeros((_L,), jnp.float32)

        def fix_chunk(c, carry):
            ids = idx_v[pl.ds(c * _L, _L)]
            pad = ids == 0
            npad = jnp.sum(pad.astype(jnp.int32))

            @pl.when(npad > 0)
            def _():
                rows = c * _L + lax.iota(jnp.int32, _L)
                for col in range(D):
                    plsc.store_scatter(
                        rows_v,
                        [rows, jnp.full((_L,), col, jnp.int32)],
                        zeros,
                        mask=pad,
                    )

            return carry

        lax.fori_loop(0, BPW // _L, fix_chunk, 0)
        pltpu.sync_copy(rows_v, out_hbm.at[b, pl.ds(t0, BPW)])

    return lookup


def kernel(inputs, embedding_matrix):
    nb, ctx = inputs.shape
    _, D = embedding_matrix.shape
    return _make_lookup(nb, ctx, D)(embedding_matrix, inputs)


# speedup vs baseline: 1.4544x; 1.1481x over previous
import functools

import jax
import jax.numpy as jnp
from jax import lax
from jax.experimental import pallas as pl
from jax.experimental.pallas import tpu as pltpu
from jax.experimental.pallas import tpu_sc as plsc

_INFO = plsc.get_sparse_core_info()
_NC, _NS, _L = _INFO.num_cores, _INFO.num_subcores, _INFO.num_lanes
_NW = _NC * _NS


def _make_lookup(NB, CTX, D):
    B = NB * CTX
    BPW = B // _NW
    WPB = _NW // NB
    K = 16
    mesh = plsc.VectorSubcoreMesh(core_axis_name="c", subcore_axis_name="s")

    @functools.partial(
        pl.kernel,
        mesh=mesh,
        compiler_params=pltpu.CompilerParams(
            use_tc_tiling_on_sc=True, needs_layout_passes=False),
        out_type=jax.ShapeDtypeStruct((NB, CTX, D), jnp.float32),
        scratch_types=[
            pltpu.VMEM((BPW,), jnp.int32),
            pltpu.VMEM((BPW, D), jnp.float32),
            pltpu.SemaphoreType.DMA,
        ],
    )
    def lookup(table_hbm, idx_hbm, out_hbm, idx_v, rows_v, sem):
        wid = lax.axis_index("s") * _NC + lax.axis_index("c")
        b = wid // WPB
        t0 = (wid % WPB) * BPW
        pltpu.sync_copy(idx_hbm.at[b, pl.ds(t0, BPW)], idx_v)

        def fire_group(g, carry):
            chunk = idx_v[pl.ds(g * K, K)]
            for j in range(K):
                pltpu.async_copy(
                    table_hbm.at[pl.ds(chunk[j], 1)],
                    rows_v.at[pl.ds(g * K + j, 1)],
                    sem,
                )
            return carry

        lax.fori_loop(0, BPW // K, fire_group, 0)

        def drain_group(g, carry):
            for j in range(K):
                pltpu.make_async_copy(
                    table_hbm.at[pl.ds(0, 1)],
                    rows_v.at[pl.ds(g * K + j, 1)],
                    sem,
                ).wait()
            return carry

        lax.fori_loop(0, BPW // K, drain_group, 0)

        zeros = jnp.zeros((_L,), jnp.float32)

        def fix_chunk(c, carry):
            ids = idx_v[pl.ds(c * _L, _L)]
            pad = ids == 0
            npad = jnp.sum(pad.astype(jnp.int32))

            @pl.when(npad > 0)
            def _():
                rows = c * _L + lax.iota(jnp.int32, _L)

                def zero_col(col, carry2):
                    plsc.store_scatter(
                        rows_v,
                        [rows, jnp.full((_L,), 0, jnp.int32) + col],
                        zeros,
                        mask=pad,
                    )
                    return carry2

                lax.fori_loop(0, D, zero_col, 0)

            return carry

        lax.fori_loop(0, BPW // _L, fix_chunk, 0)
        pltpu.sync_copy(rows_v, out_hbm.at[b, pl.ds(t0, BPW)])

    return lookup


def kernel(inputs, embedding_matrix):
    nb, ctx = inputs.shape
    _, D = embedding_matrix.shape
    return _make_lookup(nb, ctx, D)(embedding_matrix, inputs)
